# Initial kernel scaffold; baseline (speedup 1.0000x reference)
import jax
import jax.numpy as jnp
from jax.experimental import pallas as pl


def kernel(edge_index, label, train_idx):
    raise NotImplementedError("write your pallas kernel here")


# speedup vs baseline: 4.8989x; 4.8989x over previous
import functools

import jax
import jax.numpy as jnp
from jax import lax
from jax.experimental import pallas as pl
from jax.experimental.pallas import tpu as pltpu
from jax.experimental.pallas import tpu_sc as plsc

N = 10000
C = 128
E = 320000
ALPHA = 0.9
NUM_ITERS = 10

NC = 2
NS = 16
NW = NC * NS
EPW = E // NW
CHUNK = 128
NCH = -(-EPW // CHUNK)
EPAD = NCH * CHUNK
ROWS_PAD = 10240
TPW = ROWS_PAD // NW
TPS = ROWS_PAD // NS
ZROWS = 64

_MESH = plsc.VectorSubcoreMesh(core_axis_name="c", subcore_axis_name="s")


def _fori(n, body):
    lax.fori_loop(jnp.int32(0), jnp.int32(n), body, 0)


@functools.partial(
    pl.kernel,
    out_type=jax.ShapeDtypeStruct((NC, ROWS_PAD, C), jnp.float32),
    mesh=_MESH,
    scratch_types=[
        pltpu.VMEM((NCH, CHUNK), jnp.int32),
        pltpu.VMEM((NCH, CHUNK), jnp.int32),
        pltpu.VMEM((CHUNK, C), jnp.float32),
        pltpu.VMEM((ZROWS, C), jnp.float32),
        pltpu.VMEM_SHARED((ROWS_PAD, C), jnp.float32),
        pltpu.SemaphoreType.DMA,
    ],
)
def _spmm(xs_hbm, rowp_hbm, colp_hbm, out_hbm, rowi, coli, gbuf, zbuf, acc, sem):
    cid = lax.axis_index("c")
    sid = lax.axis_index("s")
    w = cid * NS + sid

    pltpu.sync_copy(rowp_hbm.at[w], rowi)
    pltpu.sync_copy(colp_hbm.at[w], coli)

    def _zrow(r, carry):
        for k in range(C // 16):
            zbuf[r, pl.ds(k * 16, 16)] = jnp.zeros((16,), jnp.float32)
        return carry

    _fori(ZROWS, _zrow)

    zbase = sid * TPS

    def _zacc(i, carry):
        pltpu.sync_copy(zbuf, acc.at[pl.ds(zbase + i * ZROWS, ZROWS)])
        return carry

    _fori(TPS // ZROWS, _zacc)
    plsc.subcore_barrier()

    def _edge(j, carry):
        pltpu.async_copy(xs_hbm.at[rowi.at[j]], gbuf, sem).wait()
        pltpu.sync_copy(gbuf, acc.at[coli.at[j]], add=True)
        return carry

    _fori(NCH, _edge)
    plsc.subcore_barrier()

    pltpu.sync_copy(acc.at[pl.ds(zbase, TPS)], out_hbm.at[cid, pl.ds(zbase, TPS)])


@functools.partial(
    pl.kernel,
    out_type=jax.ShapeDtypeStruct((ROWS_PAD, C), jnp.float32),
    mesh=_MESH,
    scratch_types=[
        pltpu.VMEM((TPW, C), jnp.float32),
        pltpu.VMEM((TPW, C), jnp.float32),
        pltpu.VMEM((TPW, C), jnp.float32),
        pltpu.VMEM((TPW,), jnp.float32),
    ],
    compiler_params=pltpu.CompilerParams(needs_layout_passes=False),
)
def _combine(part_hbm, scale_hbm, add_hbm, out_hbm, a0, a1, ab, sv):
    w = lax.axis_index("c") * NS + lax.axis_index("s")
    base = w * TPW
    pltpu.sync_copy(part_hbm.at[jnp.int32(0), pl.ds(base, TPW)], a0)
    pltpu.sync_copy(part_hbm.at[jnp.int32(1), pl.ds(base, TPW)], a1)
    pltpu.sync_copy(add_hbm.at[pl.ds(base, TPW)], ab)
    pltpu.sync_copy(scale_hbm.at[pl.ds(base, TPW)], sv)

    def _row(r, carry):
        sc = plsc.load_gather(sv, [jnp.zeros((16,), jnp.int32) + r])
        for k in range(C // 16):
            s = pl.ds(k * 16, 16)
            a0[r, s] = sc * (a0[r, s] + a1[r, s]) + ab[r, s]
        return carry

    _fori(TPW, _row)
    pltpu.sync_copy(a0, out_hbm.at[pl.ds(base, TPW)])


def kernel(edge_index, label, train_idx):
    row = edge_index[0].astype(jnp.int32)
    col = edge_index[1].astype(jnp.int32)
    label = label.astype(jnp.float32)
    ti = train_idx.astype(jnp.int32)

    deg = jnp.zeros((N,), jnp.float32).at[col].add(1.0)
    dis = jnp.where(deg > 0, lax.rsqrt(jnp.maximum(deg, 1.0)), 0.0)
    y = jnp.zeros((N, C), jnp.float32).at[ti].set(label[ti])

    rowp = jnp.pad(row.reshape(NW, EPW), ((0, 0), (0, EPAD - EPW)),
                   constant_values=0).reshape(NW, NCH, CHUNK)
    colp = jnp.pad(col.reshape(NW, EPW), ((0, 0), (0, EPAD - EPW)),
                   constant_values=N).reshape(NW, NCH, CHUNK)

    d2 = dis * dis
    pad1 = (0, ROWS_PAD - N)
    scale_h1 = jnp.pad(d2, pad1)
    scale_h2 = ALPHA * scale_h1
    scale_fin = ALPHA * jnp.pad(dis, pad1)
    add_zero = jnp.zeros((ROWS_PAD, C), jnp.float32)
    yb = jnp.pad((1.0 - ALPHA) * dis[:, None] * y, (pad1, (0, 0)))
    yfin = jnp.pad((1.0 - ALPHA) * y, (pad1, (0, 0)))
    xs = jnp.pad(dis[:, None] * y, (pad1, (0, 0)))

    for i in range(NUM_ITERS):
        part = _spmm(xs, rowp, colp)
        xs = _combine(part, scale_h1, add_zero)
        part = _spmm(xs, rowp, colp)
        if i < NUM_ITERS - 1:
            xs = _combine(part, scale_h2, yb)
        else:
            out = _combine(part, scale_fin, yfin)
    return out[:N]
